# Initial kernel scaffold; baseline (speedup 1.0000x reference)
import jax
import jax.numpy as jnp
from jax.experimental import pallas as pl


def kernel(x, target, m_list):
    raise NotImplementedError("write your pallas kernel here")


# speedup vs baseline: 1.3529x; 1.3529x over previous
import functools

import jax
import jax.numpy as jnp
from jax import lax
from jax.experimental import pallas as pl
from jax.experimental.pallas import tpu as pltpu
from jax.experimental.pallas import tpu_sc as plsc

_B = 16384
_C = 16
_S = 30.0
_LN2 = 0.6931471805599453


def _make_sc_call(num_cores: int, num_subcores: int):
    nw = num_cores * num_subcores
    rows_per_w = _B // nw
    strips = rows_per_w // 16
    blk = rows_per_w * _C

    mesh = plsc.VectorSubcoreMesh(core_axis_name="c", subcore_axis_name="s")

    @functools.partial(
        pl.kernel,
        mesh=mesh,
        compiler_params=pltpu.CompilerParams(needs_layout_passes=False),
        out_type=jax.ShapeDtypeStruct((nw * 16,), jnp.float32),
        scratch_types=[
            pltpu.VMEM((blk,), jnp.float32),
            pltpu.VMEM((blk,), jnp.int32),
            pltpu.VMEM((_C,), jnp.float32),
            pltpu.VMEM((16,), jnp.float32),
        ],
    )
    def ldam_sc(x_hbm, t_hbm, m_hbm, out_hbm, xv, tv, mv, stage):
        wid = lax.axis_index("s") * num_cores + lax.axis_index("c")
        base = wid * blk
        pltpu.sync_copy(x_hbm.at[pl.ds(base, blk)], xv)
        pltpu.sync_copy(t_hbm.at[pl.ds(base, blk)], tv)
        pltpu.sync_copy(m_hbm, mv)

        lane16 = lax.iota(jnp.int32, 16) * _C

        def strip(i, acc):
            idx0 = i * (16 * _C) + lane16
            c = plsc.load_gather(tv, [idx0 + 15]) * 2
            ms = plsc.load_gather(mv, [c]) * _S
            zero = jnp.zeros((16,), jnp.float32)
            d = zero
            ssum = zero
            logits = []
            for j in range(_C):
                xj = plsc.load_gather(xv, [idx0 + j])
                tj = plsc.load_gather(tv, [idx0 + j])
                lj = xj * _S - jnp.where(c == j, ms, zero)
                logits.append(lj)
                t2f = (tj * 2).astype(jnp.float32)
                d = d + t2f * lj
                ssum = ssum + t2f
            mx = logits[0]
            for j in range(1, _C):
                mx = jnp.maximum(mx, logits[j])
            z = zero
            for j in range(_C):
                z = z + jnp.exp(logits[j] - mx)
            zi = lax.bitcast_convert_type(z, jnp.int32)
            e = (lax.shift_right_logical(zi, 23) - 127).astype(jnp.float32)
            mant = lax.bitcast_convert_type(
                (zi & 0x007FFFFF) | 0x3F800000, jnp.float32
            )
            a = (mant - 1.0) / (mant + 1.0)
            a2 = a * a
            y = e * _LN2 + 2.0 * a * (1.0 + a2 * (1.0 / 3.0 + a2 * 0.2))
            y = y + z * jnp.exp(-y) - 1.0
            y = y + z * jnp.exp(-y) - 1.0
            lse = mx + y
            return acc + (ssum * lse - d)

        acc = lax.fori_loop(0, strips, strip, jnp.zeros((16,), jnp.float32))
        stage[...] = acc * (1.0 / _B)
        pltpu.sync_copy(stage, out_hbm.at[pl.ds(wid * 16, 16)])

    return ldam_sc


def kernel(x, target, m_list):
    info = plsc.get_sparse_core_info()
    sc_call = _make_sc_call(info.num_cores, info.num_subcores)
    partials = sc_call(x.reshape(-1), target.reshape(-1), m_list[15])
    return jnp.sum(partials)
